# baseline (speedup 1.0000x reference)
import numpy as np
import jax
import jax.numpy as jnp
from jax import lax
from jax.experimental import pallas as pl
from jax.experimental.pallas import tpu as pltpu

_FACTOR = 3
_IGNORE = -100
_N = 262144
_C = 21
_SUP = _C * 128
_NSUP = _N // 128
_SB = 128
_G = _NSUP // _SB

_A_NP = (np.arange(_SUP)[:, None] // _C == np.arange(128)[None, :]).astype(np.float32)
_CLASSMOD_NP = (np.arange(_SUP) % _C).astype(np.float32).reshape(1, _SUP)


def _flat_body(pred_ref, lab_ref, amat_ref, cmat_ref, clsm_ref,
               pos_ref, lse_ref, picked_ref):
    x = pred_ref[...]
    lab = lab_ref[...]
    amat = amat_ref[...]
    cmat = cmat_ref[...]
    clsm = clsm_ref[...]

    m = jnp.max(x)
    e = jnp.exp(x - m)
    s = jnp.dot(e, amat, preferred_element_type=jnp.float32)
    lse_sum = jnp.sum(jnp.log(s)) + m * (_SB * 128)

    labf = lab.astype(jnp.float32)
    labsp = jnp.dot(labf, cmat, preferred_element_type=jnp.float32)
    picked_sum = jnp.sum(jnp.where(labsp == clsm, x, 0.0))

    i = pl.program_id(0)
    pos_ref[0, i] = jnp.sum((lab != 0).astype(jnp.int32))
    lse_ref[0, i] = lse_sum
    picked_ref[0, i] = picked_sum


def _ce_pass(pred, label):
    pred2 = pred.reshape(_NSUP, _SUP)
    lab2 = label.reshape(_NSUP, 128)
    out = pl.pallas_call(
        _flat_body,
        grid=(_G,),
        in_specs=[
            pl.BlockSpec((_SB, _SUP), lambda i: (i, 0)),
            pl.BlockSpec((_SB, 128), lambda i: (i, 0)),
            pl.BlockSpec((_SUP, 128), lambda i: (0, 0)),
            pl.BlockSpec((128, _SUP), lambda i: (0, 0)),
            pl.BlockSpec((1, _SUP), lambda i: (0, 0)),
        ],
        out_specs=[
            pl.BlockSpec((1, _G), lambda i: (0, 0), memory_space=pltpu.SMEM),
            pl.BlockSpec((1, _G), lambda i: (0, 0), memory_space=pltpu.SMEM),
            pl.BlockSpec((1, _G), lambda i: (0, 0), memory_space=pltpu.SMEM),
        ],
        out_shape=[
            jax.ShapeDtypeStruct((1, _G), jnp.int32),
            jax.ShapeDtypeStruct((1, _G), jnp.float32),
            jax.ShapeDtypeStruct((1, _G), jnp.float32),
        ],
    )(pred2, lab2, jnp.asarray(_A_NP), jnp.asarray(_A_NP.T), jnp.asarray(_CLASSMOD_NP))
    pos_parts, lse_parts, picked_parts = out
    return jnp.sum(pos_parts), jnp.sum(lse_parts), jnp.sum(picked_parts)


def _rare_ohem_branch(ops):
    pred, label, pos_num, neg_count, neg_sum = ops
    pred_value = jnp.max(pred[:, 1:], axis=1)
    is_neg = label == 0
    padded = jnp.where(is_neg, -pred_value, jnp.inf)
    sorted_neg_score = jnp.sort(padded)
    raw_idx = neg_sum - 1
    idx = jnp.where(raw_idx >= 0, raw_idx, neg_count + raw_idx)
    idx = jnp.clip(idx, 0, padded.shape[0] - 1)
    threshold = -sorted_neg_score[idx]
    mask = (pred_value >= threshold) | (label != 0)
    masked_label = jnp.where(mask, label, _IGNORE)
    logp = jax.nn.log_softmax(pred, axis=1)
    valid = masked_label != _IGNORE
    safe = jnp.where(valid, masked_label, 0)
    nll = -jnp.take_along_axis(logp, safe[:, None], axis=1)[:, 0]
    denom = jnp.maximum(jnp.sum(valid), 1).astype(pred.dtype)
    return jnp.sum(jnp.where(valid, nll, 0.0)) / denom


def kernel(pred, label):
    pos_num, sum_lse, sum_picked = _ce_pass(pred, label)
    neg_count = _N - pos_num
    neg_sum = pos_num * _FACTOR
    common = (sum_lse - sum_picked) / jnp.float32(_N)
    return lax.cond(
        neg_count > neg_sum,
        _rare_ohem_branch,
        lambda ops: common,
        (pred, label, pos_num, neg_count, neg_sum),
    )


# speedup vs baseline: 3.9882x; 3.9882x over previous
import jax
import jax.numpy as jnp
from jax import lax
from jax.experimental import pallas as pl
from jax.experimental.pallas import tpu as pltpu

_FACTOR = 3
_IGNORE = -100
_N = 262144
_C = 21
_BLK = 32768
_G = _N // _BLK


def _ce_pass_body(pred_ref, label_ref, pos_ref, lse_ref, picked_ref):
    x = pred_ref[...]
    lab = label_ref[0, 0, :]
    m = jnp.max(x)
    e = jnp.exp(x - m)
    s = jnp.sum(e, axis=0)
    lse = jnp.log(s) + m
    cls = lax.broadcasted_iota(jnp.int32, x.shape, 0)
    picked = jnp.sum(jnp.where(cls == lab[None, :], x, 0.0), axis=0)
    i = pl.program_id(0)
    pos_ref[0, i] = jnp.sum((lab != 0).astype(jnp.int32))
    lse_ref[0, i] = jnp.sum(lse)
    picked_ref[0, i] = jnp.sum(picked)


def _ce_pass(pred, label):
    pred_t = pred.T
    label3 = label.reshape(_G, 1, _BLK)
    out = pl.pallas_call(
        _ce_pass_body,
        grid=(_G,),
        in_specs=[
            pl.BlockSpec((_C, _BLK), lambda i: (0, i)),
            pl.BlockSpec((1, 1, _BLK), lambda i: (i, 0, 0)),
        ],
        out_specs=[
            pl.BlockSpec((1, _G), lambda i: (0, 0), memory_space=pltpu.SMEM),
            pl.BlockSpec((1, _G), lambda i: (0, 0), memory_space=pltpu.SMEM),
            pl.BlockSpec((1, _G), lambda i: (0, 0), memory_space=pltpu.SMEM),
        ],
        out_shape=[
            jax.ShapeDtypeStruct((1, _G), jnp.int32),
            jax.ShapeDtypeStruct((1, _G), jnp.float32),
            jax.ShapeDtypeStruct((1, _G), jnp.float32),
        ],
    )(pred_t, label3)
    pos_parts, lse_parts, picked_parts = out
    return jnp.sum(pos_parts), jnp.sum(lse_parts), jnp.sum(picked_parts)


def _rare_ohem_branch(ops):
    pred, label, pos_num, neg_count, neg_sum = ops
    pred_value = jnp.max(pred[:, 1:], axis=1)
    is_neg = label == 0
    padded = jnp.where(is_neg, -pred_value, jnp.inf)
    sorted_neg_score = jnp.sort(padded)
    raw_idx = neg_sum - 1
    idx = jnp.where(raw_idx >= 0, raw_idx, neg_count + raw_idx)
    idx = jnp.clip(idx, 0, padded.shape[0] - 1)
    threshold = -sorted_neg_score[idx]
    mask = (pred_value >= threshold) | (label != 0)
    masked_label = jnp.where(mask, label, _IGNORE)
    logp = jax.nn.log_softmax(pred, axis=1)
    valid = masked_label != _IGNORE
    safe = jnp.where(valid, masked_label, 0)
    nll = -jnp.take_along_axis(logp, safe[:, None], axis=1)[:, 0]
    denom = jnp.maximum(jnp.sum(valid), 1).astype(pred.dtype)
    return jnp.sum(jnp.where(valid, nll, 0.0)) / denom


def kernel(pred, label):
    pos_num, sum_lse, sum_picked = _ce_pass(pred, label)
    neg_count = _N - pos_num
    neg_sum = pos_num * _FACTOR
    common = (sum_lse - sum_picked) / jnp.float32(_N)
    return lax.cond(
        neg_count > neg_sum,
        _rare_ohem_branch,
        lambda ops: common,
        (pred, label, pos_num, neg_count, neg_sum),
    )
